# baseline (speedup 1.0000x reference)
import functools

import jax
import jax.numpy as jnp
from jax import lax
from jax.experimental import pallas as pl
from jax.experimental.pallas import tpu as pltpu
from jax.experimental.pallas import tpu_sc as plsc

_LANES = 16
_NUM_WORKERS = 32


@functools.cache
def _build(B, F, VOCAB, FD):
    bpw = B // _NUM_WORKERS
    assert B % (_NUM_WORKERS * _LANES) == 0
    n_chunks = bpw // _LANES

    mesh = plsc.VectorSubcoreMesh(core_axis_name="c", subcore_axis_name="s")

    @functools.partial(
        pl.kernel,
        mesh=mesh,
        out_type=jax.ShapeDtypeStruct((B,), jnp.float32),
        compiler_params=pltpu.CompilerParams(
            needs_layout_passes=False, use_tc_tiling_on_sc=False),
        scratch_types=[
            pltpu.VMEM((bpw, F), jnp.int32),
            pltpu.VMEM((F, bpw), jnp.int32),
            pltpu.VMEM((F, bpw), jnp.float32),
            pltpu.VMEM((bpw, FD), jnp.float32),
            pltpu.VMEM((_LANES,), jnp.float32),
            pltpu.VMEM((bpw,), jnp.float32),
            pltpu.SemaphoreType.DMA,
            pltpu.SemaphoreType.DMA,
            pltpu.SemaphoreType.DMA,
        ],
    )
    def k(xs_hbm, table_hbm, xd_hbm, w_hbm, out_hbm,
          blk_v, idx_v, val_v, xd_v, w_v, acc_v, sem_i, sem_d, sem_g):
        wid = lax.axis_index("s") * 2 + lax.axis_index("c")
        base = wid * bpw

        cp_idx = pltpu.async_copy(xs_hbm.at[pl.ds(base, bpw), :], blk_v, sem_i)
        cp_xd = pltpu.async_copy(xd_hbm.at[pl.ds(base, bpw), :], xd_v, sem_d)
        cp_w = pltpu.async_copy(w_hbm, w_v.at[pl.ds(0, FD)], sem_d)

        iota = lax.iota(jnp.int32, _LANES)
        rows = [iota + jnp.full((_LANES,), c * _LANES, jnp.int32)
                for c in range(n_chunks)]

        cp_idx.wait()
        gathers = []
        for f in range(F):
            col = jnp.full((_LANES,), f, jnp.int32)
            for c in range(n_chunks):
                idx_v[f, pl.ds(c * _LANES, _LANES)] = plsc.load_gather(
                    blk_v, [rows[c], col])
            gathers.append(
                pltpu.async_copy(table_hbm.at[f].at[idx_v.at[f]], val_v.at[f],
                                 sem_g))

        cp_xd.wait()
        cp_w.wait()
        wv = w_v[:]
        w_splat = [
            jnp.broadcast_to(jnp.sum(jnp.where(iota == d, wv, 0.0)), (_LANES,))
            for d in range(FD)
        ]
        for c in range(n_chunks):
            s = w_splat[0] * plsc.load_gather(
                xd_v, [rows[c], jnp.full((_LANES,), 0, jnp.int32)])
            for d in range(1, FD):
                s = s + w_splat[d] * plsc.load_gather(
                    xd_v, [rows[c], jnp.full((_LANES,), d, jnp.int32)])
            acc_v[pl.ds(c * _LANES, _LANES)] = s

        for cp in gathers:
            cp.wait()

        for c in range(n_chunks):
            sl = pl.ds(c * _LANES, _LANES)
            s = acc_v[sl]
            for f in range(F):
                s = s + val_v[f, sl]
            acc_v[sl] = s

        pltpu.sync_copy(acc_v, out_hbm.at[pl.ds(base, bpw)])

    return k


def kernel(x_sparse, x_dense, table, W_dense):
    F, VOCAB = table.shape
    B, FD = x_dense.shape
    out = _build(B, F, VOCAB, FD)(
        x_sparse.astype(jnp.int32), table, x_dense, W_dense.reshape(-1))
    return out.reshape(B, 1)


# speedup vs baseline: 1.0426x; 1.0426x over previous
import functools

import jax
import jax.numpy as jnp
from jax import lax
from jax.experimental import pallas as pl
from jax.experimental.pallas import tpu as pltpu
from jax.experimental.pallas import tpu_sc as plsc

_LANES = 16
_NUM_WORKERS = 32


@functools.cache
def _build(B, F, VOCAB, FD):
    bpw = B // _NUM_WORKERS
    assert B % (_NUM_WORKERS * _LANES) == 0
    n_chunks = bpw // _LANES

    mesh = plsc.VectorSubcoreMesh(core_axis_name="c", subcore_axis_name="s")

    @functools.partial(
        pl.kernel,
        mesh=mesh,
        out_type=jax.ShapeDtypeStruct((B,), jnp.float32),
        compiler_params=pltpu.CompilerParams(
            needs_layout_passes=False, use_tc_tiling_on_sc=False),
        scratch_types=[
            pltpu.VMEM((F, bpw), jnp.int32),
            pltpu.VMEM((F, bpw), jnp.float32),
            pltpu.VMEM((FD, bpw), jnp.float32),
            pltpu.VMEM((_LANES,), jnp.float32),
            pltpu.SemaphoreType.DMA,
            pltpu.SemaphoreType.DMA,
            pltpu.SemaphoreType.DMA,
        ],
    )
    def k(idx_hbm, table_hbm, xd_hbm, w_hbm, out_hbm,
          idx_v, val_v, xd_v, w_v, sem_i, sem_d, sem_g):
        wid = lax.axis_index("s") * 2 + lax.axis_index("c")
        base = wid * bpw

        cp_idx = pltpu.async_copy(idx_hbm.at[:, pl.ds(base, bpw)], idx_v, sem_i)
        cp_xd = pltpu.async_copy(xd_hbm.at[:, pl.ds(base, bpw)], xd_v, sem_d)
        cp_w = pltpu.async_copy(w_hbm, w_v.at[pl.ds(0, FD)], sem_d)

        cp_idx.wait()
        gathers = [
            pltpu.async_copy(table_hbm.at[f].at[idx_v.at[f]], val_v.at[f],
                             sem_g)
            for f in range(F)
        ]

        cp_xd.wait()
        cp_w.wait()
        iota = lax.iota(jnp.int32, _LANES)
        wv = w_v[:]
        w_splat = [
            jnp.broadcast_to(jnp.sum(jnp.where(iota == d, wv, 0.0)), (_LANES,))
            for d in range(FD)
        ]
        acc = []
        for c in range(n_chunks):
            sl = pl.ds(c * _LANES, _LANES)
            s = w_splat[0] * xd_v[0, sl]
            for d in range(1, FD):
                s = s + w_splat[d] * xd_v[d, sl]
            acc.append(s)

        for f in range(F):
            gathers[f].wait()
            for c in range(n_chunks):
                acc[c] = acc[c] + val_v[f, pl.ds(c * _LANES, _LANES)]

        for c in range(n_chunks):
            val_v[0, pl.ds(c * _LANES, _LANES)] = acc[c]
        pltpu.sync_copy(val_v.at[0], out_hbm.at[pl.ds(base, bpw)])

    return k


def kernel(x_sparse, x_dense, table, W_dense):
    F, VOCAB = table.shape
    B, FD = x_dense.shape
    out = _build(B, F, VOCAB, FD)(
        x_sparse.T.astype(jnp.int32),
        table,
        x_dense.T,
        W_dense.reshape(-1),
    )
    return out.reshape(B, 1)


# speedup vs baseline: 1.0499x; 1.0070x over previous
import functools

import jax
import jax.numpy as jnp
from jax import lax
from jax.experimental import pallas as pl
from jax.experimental.pallas import tpu as pltpu
from jax.experimental.pallas import tpu_sc as plsc

_LANES = 16
_NUM_WORKERS = 32


@functools.cache
def _build(B, F, VOCAB, FD):
    bpw = B // _NUM_WORKERS
    assert B % (_NUM_WORKERS * _LANES) == 0
    n_chunks = bpw // _LANES

    mesh = plsc.VectorSubcoreMesh(core_axis_name="c", subcore_axis_name="s")

    @functools.partial(
        pl.kernel,
        mesh=mesh,
        out_type=jax.ShapeDtypeStruct((B,), jnp.float32),
        compiler_params=pltpu.CompilerParams(
            needs_layout_passes=False, use_tc_tiling_on_sc=False),
        scratch_types=[
            pltpu.VMEM((F, bpw), jnp.int32),
            pltpu.VMEM((F, bpw), jnp.float32),
            pltpu.VMEM((FD, bpw), jnp.float32),
            pltpu.VMEM((_LANES,), jnp.float32),
            pltpu.SemaphoreType.DMA,
            pltpu.SemaphoreType.DMA,
            pltpu.SemaphoreType.DMA,
        ],
    )
    def k(idx_hbm, table_hbm, xd_hbm, w_hbm, out_hbm,
          idx_v, val_v, xd_v, w_v, sem_i, sem_d, sem_g):
        wid = lax.axis_index("s") * 2 + lax.axis_index("c")
        base = wid * bpw

        cp_idx = pltpu.async_copy(idx_hbm.at[:, pl.ds(base, bpw)], idx_v, sem_i)
        cp_xd = pltpu.async_copy(xd_hbm.at[:, pl.ds(base, bpw)], xd_v, sem_d)
        cp_w = pltpu.async_copy(w_hbm, w_v.at[pl.ds(0, FD)], sem_d)

        cp_idx.wait()
        gathers = [
            pltpu.async_copy(table_hbm.at[f].at[idx_v.at[f]], val_v.at[f],
                             sem_g)
            for f in range(F)
        ]

        cp_xd.wait()
        cp_w.wait()
        iota = lax.iota(jnp.int32, _LANES)
        wv = w_v[:]
        w_splat = [
            jnp.broadcast_to(jnp.sum(jnp.where(iota == d, wv, 0.0)), (_LANES,))
            for d in range(FD)
        ]
        acc = []
        for c in range(n_chunks):
            sl = pl.ds(c * _LANES, _LANES)
            s = w_splat[0] * xd_v[0, sl]
            for d in range(1, FD):
                s = s + w_splat[d] * xd_v[d, sl]
            acc.append(s)

        for cp in gathers:
            cp.wait()
        for f in range(F):
            for c in range(n_chunks):
                acc[c] = acc[c] + val_v[f, pl.ds(c * _LANES, _LANES)]

        for c in range(n_chunks):
            val_v[0, pl.ds(c * _LANES, _LANES)] = acc[c]
        pltpu.sync_copy(val_v.at[0], out_hbm.at[pl.ds(base, bpw)])

    return k


def kernel(x_sparse, x_dense, table, W_dense):
    F, VOCAB = table.shape
    B, FD = x_dense.shape
    out = _build(B, F, VOCAB, FD)(
        x_sparse.T.astype(jnp.int32),
        table,
        x_dense.T,
        W_dense.reshape(-1),
    )
    return out.reshape(B, 1)


# speedup vs baseline: 1.0702x; 1.0194x over previous
import functools

import jax
import jax.numpy as jnp
from jax import lax
from jax.experimental import pallas as pl
from jax.experimental.pallas import tpu as pltpu
from jax.experimental.pallas import tpu_sc as plsc

_LANES = 16
_NUM_WORKERS = 32


@functools.cache
def _build(B, F, VOCAB, FD):
    bpw = B // _NUM_WORKERS
    assert B % (_NUM_WORKERS * _LANES) == 0
    n_chunks = bpw // _LANES

    mesh = plsc.VectorSubcoreMesh(core_axis_name="c", subcore_axis_name="s")

    @functools.partial(
        pl.kernel,
        mesh=mesh,
        out_type=jax.ShapeDtypeStruct((B,), jnp.float32),
        scratch_types=[
            pltpu.VMEM((F, bpw), jnp.int32),
            pltpu.VMEM((F, bpw), jnp.float32),
            pltpu.VMEM((FD, bpw), jnp.float32),
            pltpu.VMEM((FD, _LANES), jnp.float32),
            pltpu.VMEM((bpw,), jnp.float32),
            pltpu.SemaphoreType.DMA,
        ],
    )
    def k(idx_hbm, tflat_hbm, xd_hbm, w_hbm, out_hbm,
          idx_v, val_v, xd_v, w_v, acc_v, sem):
        wid = lax.axis_index("s") * 2 + lax.axis_index("c")
        base = wid * bpw

        pltpu.sync_copy(idx_hbm.at[:, pl.ds(base, bpw)], idx_v)
        pltpu.sync_copy(xd_hbm.at[:, pl.ds(base, bpw)], xd_v)
        pltpu.sync_copy(w_hbm, w_v)

        for f in range(1, F):
            off = jnp.full((_LANES,), f * VOCAB, jnp.int32)
            for c in range(n_chunks):
                sl = (f, pl.ds(c * _LANES, _LANES))
                idx_v[sl] = idx_v[sl] + off

        copies = [
            pltpu.async_copy(tflat_hbm.at[idx_v.at[f]], val_v.at[f], sem)
            for f in range(F)
        ]
        for cp in copies:
            cp.wait()

        w_splat = [w_v[d, :] for d in range(FD)]

        for c in range(n_chunks):
            sl = pl.ds(c * _LANES, _LANES)
            s = val_v[0, sl]
            for f in range(1, F):
                s = s + val_v[f, sl]
            for d in range(FD):
                s = s + w_splat[d] * xd_v[d, sl]
            acc_v[sl] = s

        pltpu.sync_copy(acc_v, out_hbm.at[pl.ds(base, bpw)])

    return k


def kernel(x_sparse, x_dense, table, W_dense):
    F, VOCAB = table.shape
    B, FD = x_dense.shape
    idx_t = x_sparse.T.astype(jnp.int32)
    tflat = table.reshape(-1)
    xd_t = x_dense.T
    w = jnp.broadcast_to(W_dense.reshape(FD, 1), (FD, _LANES))
    out = _build(B, F, VOCAB, FD)(idx_t, tflat, xd_t, w)
    return out.reshape(B, 1)
